# Initial kernel scaffold; baseline (speedup 1.0000x reference)
import jax
import jax.numpy as jnp
from jax.experimental import pallas as pl


def kernel(random_vector, pre_W1, pre_b1, pre_W2, pre_b2, post_W1, post_b1, post_W2, post_b2, br_W1, br_b1, br_W2, br_b2, gin_W1, gin_b1, gin_W2, gin_b2):
    raise NotImplementedError("write your pallas kernel here")


# speedup vs baseline: 13.0324x; 13.0324x over previous
import jax
import jax.numpy as jnp
from jax.experimental import pallas as pl
from jax.experimental.pallas import tpu as pltpu

N_EV = 128
F = 128
N_GLB = 16
N_LVL = 10
N_POST = 2
D_IN = F + N_GLB

E_BLK = 8


def _leaky(x):
    return jnp.where(x >= 0, x, 0.01 * x)


def _mm(a, b):
    return jax.lax.dot_general(a, b, (((1,), (0,)), ((), ())),
                               preferred_element_type=jnp.float32)


def _bcast_rows(v, n):
    r, c = v.shape
    return jnp.broadcast_to(v.reshape(r, 1, c), (r, n, c)).reshape(r * n, c)


def _body(x0_ref, pre_W1, pre_b1, pre_W2, pre_b2, post_W1, post_b1, post_W2,
          post_b2, br_W1x, br_W1g, br_b1, br_W2, br_b2, gin_W1x, gin_W1g,
          gin_b1, gin_W2, gin_b2, out_ref):
    E = E_BLK
    w = lambda r: r[...]

    x = [x0_ref[...]]

    def dyn_hlvs(xs):
        s = None
        mx = None
        for l, xl in enumerate(xs):
            f = _mm(_leaky(_mm(xl, w(pre_W1)) + w(pre_b1)), w(pre_W2)) + w(pre_b2)
            fr = f.reshape(E, 2 ** l, F)
            sl = fr.sum(axis=1)
            ml = fr.max(axis=1)
            s = sl if s is None else s + sl
            mx = ml if mx is None else jnp.maximum(mx, ml)
        mean = s * (1.0 / float(2 ** len(xs) - 1))
        cat = jnp.concatenate([mean, mx], axis=1)
        return _mm(_leaky(_mm(cat, w(post_W1)) + w(post_b1)), w(post_W2)) + w(post_b2)

    def gin(xs, g, only_last=False):
        L = len(xs) - 1
        G1 = _mm(g, w(gin_W1g))
        out = []
        for k, xk in enumerate(xs):
            if only_last and k != L:
                out.append(xk)
                continue
            nk = 2 ** k
            xin = xk
            deg = 0
            if k >= 1:
                xin = xin + _bcast_rows(xs[k - 1], 2)
                deg += 1
            if k < L:
                xin = xin + xs[k + 1].reshape(E * nk, 2, F).sum(axis=1)
                deg += 2
            t = _mm(xin, w(gin_W1x)) + _bcast_rows((1.0 + deg) * G1, nk) + w(gin_b1)
            out.append(_mm(_leaky(t), w(gin_W2)) + w(gin_b2))
        return out

    for l in range(1, N_LVL):
        g = dyn_hlvs(x)
        nl = 2 ** (l - 1)
        t = _mm(x[-1], w(br_W1x)) + _bcast_rows(_mm(g, w(br_W1g)), nl) + w(br_b1)
        ch = _mm(_leaky(t), w(br_W2)) + w(br_b2)
        x.append(ch.reshape(E * nl * 2, F))
        x = gin(x, g)
    for i in range(N_POST):
        g = dyn_hlvs(x)
        x = gin(x, g, only_last=(i == N_POST - 1))
    out_ref[...] = x[-1]


def kernel(random_vector, pre_W1, pre_b1, pre_W2, pre_b2, post_W1, post_b1,
           post_W2, post_b2, br_W1, br_b1, br_W2, br_b2, gin_W1, gin_b1,
           gin_W2, gin_b2):
    x0 = random_vector.reshape(N_EV, F)
    n_leaf = 2 ** (N_LVL - 1)

    full = lambda a: pl.BlockSpec(a.shape, lambda i: (0,) * a.ndim)
    row = lambda b: b.reshape(1, -1)

    ins = [
        x0, pre_W1, row(pre_b1), pre_W2, row(pre_b2),
        post_W1, row(post_b1), post_W2, row(post_b2),
        br_W1[:F], br_W1[F:], row(br_b1), br_W2, row(br_b2),
        gin_W1[:F], gin_W1[F:], row(gin_b1), gin_W2, row(gin_b2),
    ]
    in_specs = [pl.BlockSpec((E_BLK, F), lambda i: (i, 0))] + [full(a) for a in ins[1:]]

    out = pl.pallas_call(
        _body,
        grid=(N_EV // E_BLK,),
        in_specs=in_specs,
        out_specs=pl.BlockSpec((E_BLK * n_leaf, F), lambda i: (i, 0)),
        out_shape=jax.ShapeDtypeStruct((N_EV * n_leaf, F), jnp.float32),
        compiler_params=pltpu.CompilerParams(
            dimension_semantics=("arbitrary",)),
    )(*ins)
    return out.reshape(N_EV, n_leaf, F)[:, :, :3]


# speedup vs baseline: 14.5143x; 1.1137x over previous
import jax
import jax.numpy as jnp
from jax.experimental import pallas as pl
from jax.experimental.pallas import tpu as pltpu

N_EV = 128
F = 128
N_GLB = 16
N_LVL = 10
N_POST = 2
D_IN = F + N_GLB

E_BLK = 16


def _leaky(x):
    return jnp.where(x >= 0, x, 0.01 * x)


def _mm(a, b):
    return jax.lax.dot_general(a, b, (((1,), (0,)), ((), ())),
                               preferred_element_type=jnp.float32)


def _bcast_rows(v, n):
    r, c = v.shape
    return jnp.broadcast_to(v.reshape(r, 1, c), (r, n, c)).reshape(r * n, c)


def _body(x0_ref, pre_W1, pre_b1, pre_W2, pre_b2, post_W1, post_b1, post_W2,
          post_b2, br_W1x, br_W1g, br_b1, br_W2, br_b2, gin_W1x, gin_W1g,
          gin_b1, gin_W2, gin_b2, out_ref):
    E = E_BLK
    w = lambda r: r[...]

    x = [x0_ref[...]]

    def dyn_hlvs(xs):
        s = None
        mx = None
        for l, xl in enumerate(xs):
            f = _mm(_leaky(_mm(xl, w(pre_W1)) + w(pre_b1)), w(pre_W2)) + w(pre_b2)
            fr = f.reshape(E, 2 ** l, F)
            sl = fr.sum(axis=1)
            ml = fr.max(axis=1)
            s = sl if s is None else s + sl
            mx = ml if mx is None else jnp.maximum(mx, ml)
        mean = s * (1.0 / float(2 ** len(xs) - 1))
        cat = jnp.concatenate([mean, mx], axis=1)
        return _mm(_leaky(_mm(cat, w(post_W1)) + w(post_b1)), w(post_W2)) + w(post_b2)

    def gin(xs, g, only_last=False):
        L = len(xs) - 1
        G1 = _mm(g, w(gin_W1g))
        out = []
        for k, xk in enumerate(xs):
            if only_last and k != L:
                out.append(xk)
                continue
            nk = 2 ** k
            xin = xk
            deg = 0
            if k >= 1:
                xin = xin + _bcast_rows(xs[k - 1], 2)
                deg += 1
            if k < L:
                xin = xin + xs[k + 1].reshape(E * nk, 2, F).sum(axis=1)
                deg += 2
            t = _mm(xin, w(gin_W1x)) + _bcast_rows((1.0 + deg) * G1, nk) + w(gin_b1)
            out.append(_mm(_leaky(t), w(gin_W2)) + w(gin_b2))
        return out

    for l in range(1, N_LVL):
        g = dyn_hlvs(x)
        nl = 2 ** (l - 1)
        t = _mm(x[-1], w(br_W1x)) + _bcast_rows(_mm(g, w(br_W1g)), nl) + w(br_b1)
        ch = _mm(_leaky(t), w(br_W2)) + w(br_b2)
        x.append(ch.reshape(E * nl * 2, F))
        x = gin(x, g)
    for i in range(N_POST):
        g = dyn_hlvs(x)
        x = gin(x, g, only_last=(i == N_POST - 1))
    out_ref[...] = x[-1]


def kernel(random_vector, pre_W1, pre_b1, pre_W2, pre_b2, post_W1, post_b1,
           post_W2, post_b2, br_W1, br_b1, br_W2, br_b2, gin_W1, gin_b1,
           gin_W2, gin_b2):
    x0 = random_vector.reshape(N_EV, F)
    n_leaf = 2 ** (N_LVL - 1)

    full = lambda a: pl.BlockSpec(a.shape, lambda i: (0,) * a.ndim)
    row = lambda b: b.reshape(1, -1)

    ins = [
        x0, pre_W1, row(pre_b1), pre_W2, row(pre_b2),
        post_W1, row(post_b1), post_W2, row(post_b2),
        br_W1[:F], br_W1[F:], row(br_b1), br_W2, row(br_b2),
        gin_W1[:F], gin_W1[F:], row(gin_b1), gin_W2, row(gin_b2),
    ]
    in_specs = [pl.BlockSpec((E_BLK, F), lambda i: (i, 0))] + [full(a) for a in ins[1:]]

    out = pl.pallas_call(
        _body,
        grid=(N_EV // E_BLK,),
        in_specs=in_specs,
        out_specs=pl.BlockSpec((E_BLK * n_leaf, F), lambda i: (i, 0)),
        out_shape=jax.ShapeDtypeStruct((N_EV * n_leaf, F), jnp.float32),
        compiler_params=pltpu.CompilerParams(
            dimension_semantics=("arbitrary",),
            vmem_limit_bytes=100 * 1024 * 1024),
    )(*ins)
    return out.reshape(N_EV, n_leaf, F)[:, :, :3]


# speedup vs baseline: 22.1852x; 1.5285x over previous
import jax
import jax.numpy as jnp
from jax.experimental import pallas as pl
from jax.experimental.pallas import tpu as pltpu

N_EV = 128
F = 128
N_GLB = 16
N_LVL = 10
N_POST = 2
D_IN = F + N_GLB

E_BLK = 16


def _leaky(x):
    return jnp.where(x >= 0, x, 0.01 * x)


def _mm(a, b):
    return jax.lax.dot_general(a, b, (((1,), (0,)), ((), ())),
                               preferred_element_type=jnp.float32)


def _bcast_rows(v, n):
    r, c = v.shape
    return jnp.broadcast_to(v.reshape(r, 1, c), (r, n, c)).reshape(r * n, c)


def _body(x0_ref, pre_W1, pre_b1, pre_W2, pre_b2, post_W1, post_b1, post_W2,
          post_b2, br_W1x, br_W1g, br_b1, br_W2, br_b2, gin_W1x, gin_W1g,
          gin_b1, gin_W2, gin_b2, out_ref):
    E = E_BLK
    w = lambda r: r[...]

    x = [x0_ref[...]]

    def dyn_hlvs(xs):
        s = None
        mx = None
        for l, xl in enumerate(xs):
            f = _mm(_leaky(_mm(xl, w(pre_W1)) + w(pre_b1)), w(pre_W2)) + w(pre_b2)
            fr = f.reshape(E, 2 ** l, F)
            sl = fr.sum(axis=1)
            ml = fr.max(axis=1)
            s = sl if s is None else s + sl
            mx = ml if mx is None else jnp.maximum(mx, ml)
        mean = s * (1.0 / float(2 ** len(xs) - 1))
        cat = jnp.concatenate([mean, mx], axis=1)
        return _mm(_leaky(_mm(cat, w(post_W1)) + w(post_b1)), w(post_W2)) + w(post_b2)

    def gin(xs, g, only_last=False):
        L = len(xs) - 1
        G1 = _mm(g, w(gin_W1g))
        out = []
        for k, xk in enumerate(xs):
            if only_last and k != L:
                out.append(xk)
                continue
            nk = 2 ** k
            xin = xk
            deg = 0
            if k >= 1:
                xin = xin + _bcast_rows(xs[k - 1], 2)
                deg += 1
            if k < L:
                c = xs[k + 1].reshape(E * nk, 2 * F)
                xin = xin + c[:, :F] + c[:, F:]
                deg += 2
            t = _mm(xin, w(gin_W1x)) + _bcast_rows((1.0 + deg) * G1, nk) + w(gin_b1)
            out.append(_mm(_leaky(t), w(gin_W2)) + w(gin_b2))
        return out

    for l in range(1, N_LVL):
        g = dyn_hlvs(x)
        nl = 2 ** (l - 1)
        t = _mm(x[-1], w(br_W1x)) + _bcast_rows(_mm(g, w(br_W1g)), nl) + w(br_b1)
        ch = _mm(_leaky(t), w(br_W2)) + w(br_b2)
        x.append(ch.reshape(E * nl * 2, F))
        x = gin(x, g)
    for i in range(N_POST):
        g = dyn_hlvs(x)
        x = gin(x, g, only_last=(i == N_POST - 1))
    out_ref[...] = x[-1]


def kernel(random_vector, pre_W1, pre_b1, pre_W2, pre_b2, post_W1, post_b1,
           post_W2, post_b2, br_W1, br_b1, br_W2, br_b2, gin_W1, gin_b1,
           gin_W2, gin_b2):
    x0 = random_vector.reshape(N_EV, F)
    n_leaf = 2 ** (N_LVL - 1)

    full = lambda a: pl.BlockSpec(a.shape, lambda i: (0,) * a.ndim)
    row = lambda b: b.reshape(1, -1)

    ins = [
        x0, pre_W1, row(pre_b1), pre_W2, row(pre_b2),
        post_W1, row(post_b1), post_W2, row(post_b2),
        br_W1[:F], br_W1[F:], row(br_b1), br_W2, row(br_b2),
        gin_W1[:F], gin_W1[F:], row(gin_b1), gin_W2, row(gin_b2),
    ]
    in_specs = [pl.BlockSpec((E_BLK, F), lambda i: (i, 0))] + [full(a) for a in ins[1:]]

    out = pl.pallas_call(
        _body,
        grid=(N_EV // E_BLK,),
        in_specs=in_specs,
        out_specs=pl.BlockSpec((E_BLK * n_leaf, F), lambda i: (i, 0)),
        out_shape=jax.ShapeDtypeStruct((N_EV * n_leaf, F), jnp.float32),
        compiler_params=pltpu.CompilerParams(
            dimension_semantics=("arbitrary",),
            vmem_limit_bytes=128 * 1024 * 1024),
    )(*ins)
    return out.reshape(N_EV, n_leaf, F)[:, :, :3]
